# baseline (speedup 1.0000x reference)
import functools

import jax
import jax.numpy as jnp
from jax import lax
from jax.experimental import pallas as pl
from jax.experimental.pallas import tpu as pltpu
from jax.experimental.pallas import tpu_sc as plsc

F32 = jnp.float32
W = 160
WG = 256
WS = 128
NB = 1000
EB = 2000
EIB = 4000
CH = 80
NCHUNK = 125


def _dot(a, b):
    return jnp.dot(a, b, preferred_element_type=F32)


def _bdot(a, b):
    return jnp.dot(a.astype(jnp.bfloat16), b.astype(jnp.bfloat16),
                   preferred_element_type=F32)


def _ln(s, g, b):
    mu = jnp.mean(s, axis=1, keepdims=True)
    var = jnp.mean(jnp.square(s - mu), axis=1, keepdims=True)
    return (s - mu) / jnp.sqrt(var + 1e-5) * g + b


def _vnorm3(hx, hy, hz):
    return jnp.sqrt(jnp.clip(hx * hx + hy * hy + hz * hz, 1e-8, None))


def _vln_v(vx, vy, vz):
    vsq = jnp.clip(vx * vx + vy * vy + vz * vz, 1e-8, None)
    den = jnp.sqrt(jnp.mean(vsq, axis=1, keepdims=True))
    return vx / den, vy / den, vz / den


def _pack_rows(row):
    u = lax.bitcast_convert_type(row, jnp.uint32) + jnp.uint32(0x8000)
    return (u[:, 0:128] >> 16) | (u[:, 128:256] & jnp.uint32(0xFFFF0000))


def _unpack_rows(w):
    lo = lax.bitcast_convert_type(w << 16, F32)
    hi = lax.bitcast_convert_type(w & jnp.uint32(0xFFFF0000), F32)
    return lo, hi


def _node_init_body(s6_ref, v9_ref, g_ref, b_ref, whT_ref, wsTs_ref, wsTv_ref,
                    bs_ref, wvT_ref, wsvT_ref, bsv_ref, asrc_ref, adst_ref,
                    sv_ref, ts_ref):
    s = s6_ref[...]
    vx, vy, vz = v9_ref[:, 0:3], v9_ref[:, 3:6], v9_ref[:, 6:9]
    s = _ln(s, g_ref[...], b_ref[...])
    vx, vy, vz = _vln_v(vx, vy, vz)
    whT = whT_ref[...]
    hx, hy, hz = _dot(vx, whT), _dot(vy, whT), _dot(vz, whT)
    vn = _vnorm3(hx, hy, hz)
    spre = _dot(s, wsTs_ref[...]) + _dot(vn, wsTv_ref[...]) + bs_ref[...]
    gate = jax.nn.sigmoid(_dot(spre, wsvT_ref[...]) + bsv_ref[...])
    wvT = wvT_ref[...]
    ox, oy, oz = _dot(hx, wvT) * gate, _dot(hy, wvT) * gate, _dot(hz, wvT) * gate
    zpad = jnp.zeros((spre.shape[0], W - 148), F32)
    sv_ref[...] = jnp.concatenate([spre, ox, oy, oz, zpad], axis=1)
    gpad = jnp.zeros((spre.shape[0], WG - 248), F32)
    ts_ref[...] = _pack_rows(jnp.concatenate(
        [_dot(spre, asrc_ref[...]), _dot(spre, adst_ref[...]), ox, oy, oz, gpad],
        axis=1))


def _edge_init_body(es_ref, ev_ref, g_ref, b_ref, wh_ref, wsTs_ref, wsTv_ref,
                    bs_ref, wv_ref, wsvT_ref, bsv_ref, ef_ref):
    s = _ln(es_ref[...], g_ref[...], b_ref[...])
    vx, vy, vz = ev_ref[:, 0:1], ev_ref[:, 1:2], ev_ref[:, 2:3]
    vx, vy, vz = _vln_v(vx, vy, vz)
    wh00 = wh_ref[0, 0]
    hx, hy, hz = vx * wh00, vy * wh00, vz * wh00
    vn = _vnorm3(hx, hy, hz)
    spre = _dot(s, wsTs_ref[...]) + vn * wsTv_ref[...] + bs_ref[...]
    gate = jax.nn.sigmoid(_dot(spre, wsvT_ref[...]) + bsv_ref[...])
    wv00 = wv_ref[0, 0]
    ox, oy, oz = hx * wv00 * gate, hy * wv00 * gate, hz * wv00 * gate
    zpad = jnp.zeros((spre.shape[0], 5), F32)
    ef_ref[...] = jnp.concatenate([spre, ox, oy, oz, zpad], axis=1)


def _edge_layer_body(gs_ref, gd_ref, ef_ref,
                     wh0T_ref, wes_ref, wvn_ref, b0_ref, wv0T_ref, wsv0T_ref, bsv0_ref,
                     wh1T_ref, ws1T_ref, b1_ref, wv1T_ref, wsv1T_ref, bsv1_ref,
                     wh2T_ref, ws2T_ref, b2_ref, wv2T_ref, wsv2T_ref, bsv2_ref,
                     msgs_ref, msgv_ref):
    glo_s, ghi_s = _unpack_rows(gs_ref[...])
    glo_d, ghi_d = _unpack_rows(gd_ref[...])
    ef = ef_ref[...]
    ps = glo_s[:, 0:100]
    pd = jnp.concatenate([glo_d[:, 100:128], ghi_d[:, 0:72]], axis=1)
    vsx, vsy, vsz = ghi_s[:, 72:88], ghi_s[:, 88:104], ghi_s[:, 104:120]
    vdx, vdy, vdz = ghi_d[:, 72:88], ghi_d[:, 88:104], ghi_d[:, 104:120]
    es = ef[:, 0:32]
    evx, evy, evz = ef[:, 32:33], ef[:, 33:34], ef[:, 34:35]
    wh0T = wh0T_ref[...]
    a, bb, c = wh0T[0:16, :], wh0T[16:17, :], wh0T[17:33, :]
    hx = _bdot(vsx, a) + _bdot(evx, bb) + _bdot(vdx, c)
    hy = _bdot(vsy, a) + _bdot(evy, bb) + _bdot(vdy, c)
    hz = _bdot(vsz, a) + _bdot(evz, bb) + _bdot(vdz, c)
    vn = _vnorm3(hx, hy, hz)
    s0 = ps + pd + _bdot(es, wes_ref[...]) + _bdot(vn, wvn_ref[...]) + b0_ref[...]
    g0 = jax.nn.sigmoid(_bdot(s0, wsv0T_ref[...]) + bsv0_ref[...])
    wv0T = wv0T_ref[...]
    ux, uy, uz = _bdot(hx, wv0T) * g0, _bdot(hy, wv0T) * g0, _bdot(hz, wv0T) * g0
    m = jax.nn.relu(s0)
    wh1T = wh1T_ref[...]
    hx, hy, hz = _bdot(ux, wh1T), _bdot(uy, wh1T), _bdot(uz, wh1T)
    vn = _vnorm3(hx, hy, hz)
    ws1T = ws1T_ref[...]
    s1 = _bdot(m, ws1T[0:100, :]) + _bdot(vn, ws1T[100:116, :]) + b1_ref[...]
    g1 = jax.nn.sigmoid(_bdot(s1, wsv1T_ref[...]) + bsv1_ref[...])
    wv1T = wv1T_ref[...]
    ux, uy, uz = _bdot(hx, wv1T) * g1, _bdot(hy, wv1T) * g1, _bdot(hz, wv1T) * g1
    m = jax.nn.relu(s1)
    wh2T = wh2T_ref[...]
    hx, hy, hz = _bdot(ux, wh2T), _bdot(uy, wh2T), _bdot(uz, wh2T)
    vn = _vnorm3(hx, hy, hz)
    ws2T = ws2T_ref[...]
    s2 = _bdot(m, ws2T[0:100, :]) + _bdot(vn, ws2T[100:116, :]) + b2_ref[...]
    g2 = jax.nn.sigmoid(_bdot(s2, wsv2T_ref[...]) + bsv2_ref[...])
    wv2T = wv2T_ref[...]
    ux, uy, uz = _bdot(hx, wv2T) * g2, _bdot(hy, wv2T) * g2, _bdot(hz, wv2T) * g2
    n = s2.shape[0]
    msgs_ref[...] = jnp.concatenate(
        [s2, jnp.ones((n, 1), F32), jnp.zeros((n, 27), F32)], axis=1)
    msgv_ref[...] = jnp.concatenate(
        [ux, uy, uz, jnp.zeros((n, 80), F32)], axis=1)


def _node_update_math(parts, partv, sv, g0_ref, c0_ref,
                      fwh0T_ref, fws0Ts_ref, fws0Tv_ref, fb0_ref, fwv0T_ref,
                      fwsv0T_ref, fbsv0_ref,
                      fwh1T_ref, fws1Ts_ref, fws1Tv_ref, fb1_ref, fwv1T_ref,
                      fwsv1T_ref, fbsv1_ref, g1_ref, c1_ref):
    aggs = parts[0] + parts[1]
    aggv = partv[0] + partv[1]
    cnt = jnp.clip(aggs[:, 100:101], 1.0, None)
    s = sv[:, 0:100] + aggs[:, 0:100] / cnt
    vx = sv[:, 100:116] + aggv[:, 0:16] / cnt
    vy = sv[:, 116:132] + aggv[:, 16:32] / cnt
    vz = sv[:, 132:148] + aggv[:, 32:48] / cnt
    s = _ln(s, g0_ref[...], c0_ref[...])
    vx, vy, vz = _vln_v(vx, vy, vz)
    fwh0T = fwh0T_ref[...]
    hx, hy, hz = _dot(vx, fwh0T), _dot(vy, fwh0T), _dot(vz, fwh0T)
    vn = _vnorm3(hx, hy, hz)
    f0 = _dot(s, fws0Ts_ref[...]) + _dot(vn, fws0Tv_ref[...]) + fb0_ref[...]
    gate = jax.nn.sigmoid(_dot(f0, fwsv0T_ref[...]) + fbsv0_ref[...])
    fwv0T = fwv0T_ref[...]
    cx, cy, cz = _dot(hx, fwv0T) * gate, _dot(hy, fwv0T) * gate, _dot(hz, fwv0T) * gate
    f0 = jax.nn.relu(f0)
    fwh1T = fwh1T_ref[...]
    hx, hy, hz = _dot(cx, fwh1T), _dot(cy, fwh1T), _dot(cz, fwh1T)
    vn = _vnorm3(hx, hy, hz)
    f1 = _dot(f0, fws1Ts_ref[...]) + _dot(vn, fws1Tv_ref[...]) + fb1_ref[...]
    gate = jax.nn.sigmoid(_dot(f1, fwsv1T_ref[...]) + fbsv1_ref[...])
    fwv1T = fwv1T_ref[...]
    dx, dy, dz = _dot(hx, fwv1T) * gate, _dot(hy, fwv1T) * gate, _dot(hz, fwv1T) * gate
    s = _ln(s + f1, g1_ref[...], c1_ref[...])
    vx, vy, vz = _vln_v(vx + dx, vy + dy, vz + dz)
    return s, vx, vy, vz


def _node_update_body(parts_ref, partv_ref, sv_ref, g0_ref, c0_ref,
                      fwh0T_ref, fws0Ts_ref, fws0Tv_ref, fb0_ref, fwv0T_ref,
                      fwsv0T_ref, fbsv0_ref,
                      fwh1T_ref, fws1Ts_ref, fws1Tv_ref, fb1_ref, fwv1T_ref,
                      fwsv1T_ref, fbsv1_ref, g1_ref, c1_ref,
                      asrc_ref, adst_ref, sv2_ref, ts_ref):
    s, vx, vy, vz = _node_update_math(
        parts_ref[...], partv_ref[...], sv_ref[...], g0_ref, c0_ref,
        fwh0T_ref, fws0Ts_ref, fws0Tv_ref, fb0_ref, fwv0T_ref, fwsv0T_ref,
        fbsv0_ref, fwh1T_ref, fws1Ts_ref, fws1Tv_ref, fb1_ref, fwv1T_ref,
        fwsv1T_ref, fbsv1_ref, g1_ref, c1_ref)
    zpad = jnp.zeros((s.shape[0], W - 148), F32)
    sv2_ref[...] = jnp.concatenate([s, vx, vy, vz, zpad], axis=1)
    gpad = jnp.zeros((s.shape[0], WG - 248), F32)
    ts_ref[...] = _pack_rows(jnp.concatenate(
        [_dot(s, asrc_ref[...]), _dot(s, adst_ref[...]), vx, vy, vz, gpad], axis=1))


def _node_final_body(parts_ref, partv_ref, sv_ref, g0_ref, c0_ref,
                     fwh0T_ref, fws0Ts_ref, fws0Tv_ref, fb0_ref, fwv0T_ref,
                     fwsv0T_ref, fbsv0_ref,
                     fwh1T_ref, fws1Ts_ref, fws1Tv_ref, fb1_ref, fwv1T_ref,
                     fwsv1T_ref, fbsv1_ref, g1_ref, c1_ref,
                     og_ref, oc_ref, owhT_ref, owsTs_ref, owsTv_ref, ob_ref,
                     dwT_ref, db_ref, out_ref):
    s, vx, vy, vz = _node_update_math(
        parts_ref[...], partv_ref[...], sv_ref[...], g0_ref, c0_ref,
        fwh0T_ref, fws0Ts_ref, fws0Tv_ref, fb0_ref, fwv0T_ref, fwsv0T_ref,
        fbsv0_ref, fwh1T_ref, fws1Ts_ref, fws1Tv_ref, fb1_ref, fwv1T_ref,
        fwsv1T_ref, fbsv1_ref, g1_ref, c1_ref)
    s = _ln(s, og_ref[...], oc_ref[...])
    vx, vy, vz = _vln_v(vx, vy, vz)
    owhT = owhT_ref[...]
    hx, hy, hz = _dot(vx, owhT), _dot(vy, owhT), _dot(vz, owhT)
    vn = _vnorm3(hx, hy, hz)
    s = jax.nn.relu(_dot(s, owsTs_ref[...]) + _dot(vn, owsTv_ref[...]) + ob_ref[...])
    out_ref[...] = jax.nn.relu(_dot(s, dwT_ref[...]) + db_ref[...])


def _sc_gather(ts, src3d, dst3d):
    n_edges = src3d.shape[0] * src3d.shape[1] * src3d.shape[2]
    per_w = NCHUNK * CH
    mesh = plsc.VectorSubcoreMesh(core_axis_name="c", subcore_axis_name="s")

    @functools.partial(
        pl.kernel,
        out_type=[jax.ShapeDtypeStruct((n_edges, WG // 2), jnp.uint32),
                  jax.ShapeDtypeStruct((n_edges, WG // 2), jnp.uint32)],
        mesh=mesh,
        scratch_types=[pltpu.VMEM((NCHUNK, CH), jnp.int32),
                       pltpu.VMEM((NCHUNK, CH), jnp.int32),
                       pltpu.VMEM((CH, WG // 2), jnp.uint32),
                       pltpu.VMEM((CH, WG // 2), jnp.uint32),
                       pltpu.SemaphoreType.DMA,
                       pltpu.SemaphoreType.DMA],
    )
    def k(ts_hbm, src_hbm, dst_hbm, gs_hbm, gd_hbm,
          isrc, idst, bufs, bufd, sems, semd):
        wid = lax.axis_index("s") * 2 + lax.axis_index("c")
        base = wid * per_w
        pltpu.sync_copy(src_hbm.at[wid], isrc)
        pltpu.sync_copy(dst_hbm.at[wid], idst)

        @pl.loop(0, NCHUNK)
        def _(j):
            cs = pltpu.async_copy(ts_hbm.at[isrc.at[j]], bufs, sems)
            cd = pltpu.async_copy(ts_hbm.at[idst.at[j]], bufd, semd)
            cs.wait()
            pltpu.sync_copy(bufs, gs_hbm.at[pl.ds(base + j * CH, CH)])
            cd.wait()
            pltpu.sync_copy(bufd, gd_hbm.at[pl.ds(base + j * CH, CH)])

    return k(ts, src3d, dst3d)


def _sc_scatter(msgs, msgv, dst3d, zeros):
    n_nodes = zeros.shape[0]
    per_w = NCHUNK * CH
    rows_per_tile = n_nodes // 10
    mesh = plsc.VectorSubcoreMesh(core_axis_name="c", subcore_axis_name="s")

    @functools.partial(
        pl.kernel,
        out_type=[jax.ShapeDtypeStruct((2, n_nodes, WS), F32),
                  jax.ShapeDtypeStruct((2, n_nodes, WS), F32)],
        mesh=mesh,
        scratch_types=[pltpu.VMEM((NCHUNK, CH), jnp.int32),
                       pltpu.VMEM((CH, WS), F32),
                       pltpu.VMEM_SHARED((n_nodes, WS), F32),
                       pltpu.SemaphoreType.DMA],
    )
    def k(msgs_hbm, msgv_hbm, dst_hbm, zero_hbm, outs_hbm, outv_hbm,
          idxv, mbuf, accum, sem):
        cid = lax.axis_index("c")
        sid = lax.axis_index("s")
        wid = sid * 2 + cid
        rows = pl.ds(sid * rows_per_tile, rows_per_tile)

        @pl.when(sid < 10)
        def _():
            pltpu.sync_copy(zero_hbm.at[rows], accum.at[rows])

        plsc.subcore_barrier()
        pltpu.sync_copy(dst_hbm.at[wid], idxv)

        @pl.loop(0, NCHUNK)
        def _(j):
            pltpu.sync_copy(msgs_hbm.at[pl.ds(wid * per_w + j * CH, CH)], mbuf)
            pltpu.sync_copy(mbuf, accum.at[idxv.at[j]], add=True)

        plsc.subcore_barrier()

        @pl.when(sid < 10)
        def _():
            pltpu.sync_copy(accum.at[rows], outs_hbm.at[cid].at[rows])
            pltpu.sync_copy(zero_hbm.at[rows], accum.at[rows])

        plsc.subcore_barrier()

        @pl.loop(0, NCHUNK)
        def _(j):
            pltpu.sync_copy(msgv_hbm.at[pl.ds(wid * per_w + j * CH, CH)], mbuf)
            pltpu.sync_copy(mbuf, accum.at[idxv.at[j]], add=True)

        plsc.subcore_barrier()

        @pl.when(sid < 10)
        def _():
            pltpu.sync_copy(accum.at[rows], outv_hbm.at[cid].at[rows])

    return k(msgs, msgv, dst3d, zeros)


def _t(p):
    return jnp.asarray(p, F32).T


def _row(p):
    return jnp.asarray(p, F32).reshape(1, -1)


def _full(shape):
    return pl.BlockSpec(shape, lambda i: tuple(0 for _ in shape))


def kernel(h_V_s, h_V_v, edge_index, h_E_s, h_E_v, params):
    n = h_V_s.shape[0]
    e = h_E_s.shape[0]
    src3d = edge_index[0].astype(jnp.int32).reshape(32, NCHUNK, CH)
    dst3d = edge_index[1].astype(jnp.int32).reshape(32, NCHUNK, CH)
    v9 = jnp.swapaxes(h_V_v, 1, 2).reshape(n, 9)
    ev3 = h_E_v.reshape(e, 3)
    zeros = jnp.zeros((n, WS), F32)
    L0 = params['layers'][0]

    gvp = params['Wv_gvp']
    wsT = _t(gvp['ws']['w'])
    sv, ts = pl.pallas_call(
        _node_init_body,
        grid=(n // NB,),
        out_shape=[jax.ShapeDtypeStruct((n, W), F32),
                   jax.ShapeDtypeStruct((n, WG // 2), jnp.uint32)],
        in_specs=[
            pl.BlockSpec((NB, 6), lambda i: (i, 0)),
            pl.BlockSpec((NB, 9), lambda i: (i, 0)),
            _full((1, 6)), _full((1, 6)),
            _full((3, 16)),
            _full((6, 100)), _full((16, 100)), _full((1, 100)),
            _full((16, 16)), _full((100, 16)), _full((1, 16)),
            _full((100, 100)), _full((100, 100)),
        ],
        out_specs=[pl.BlockSpec((NB, W), lambda i: (i, 0)),
                   pl.BlockSpec((NB, WG // 2), lambda i: (i, 0))],
    )(h_V_s.astype(F32), v9.astype(F32),
      _row(params['Wv_ln']['g']), _row(params['Wv_ln']['b']),
      _t(gvp['wh']), wsT[0:6, :], wsT[6:22, :], _row(gvp['ws']['b']),
      _t(gvp['wv']), _t(gvp['wsv']['w']), _row(gvp['wsv']['b']),
      _t(L0['msg0']['ws']['w'][:, 0:100]), _t(L0['msg0']['ws']['w'][:, 132:232]))

    gvp = params['We_gvp']
    wsT = _t(gvp['ws']['w'])
    ef = pl.pallas_call(
        _edge_init_body,
        grid=(e // EIB,),
        out_shape=jax.ShapeDtypeStruct((e, 40), F32),
        in_specs=[
            pl.BlockSpec((EIB, 32), lambda i: (i, 0)),
            pl.BlockSpec((EIB, 3), lambda i: (i, 0)),
            _full((1, 32)), _full((1, 32)),
            _full((1, 1)),
            _full((32, 32)), _full((1, 32)), _full((1, 32)),
            _full((1, 1)), _full((32, 1)), _full((1, 1)),
        ],
        out_specs=pl.BlockSpec((EIB, 40), lambda i: (i, 0)),
    )(h_E_s.astype(F32), ev3.astype(F32),
      _row(params['We_ln']['g']), _row(params['We_ln']['b']),
      jnp.asarray(gvp['wh'], F32), wsT[0:32, :], wsT[32:33, :],
      _row(gvp['ws']['b']), jnp.asarray(gvp['wv'], F32),
      _t(gvp['wsv']['w']), _row(gvp['wsv']['b']))

    for li, L in enumerate(params['layers']):
        gs, gd = _sc_gather(ts, src3d, dst3d)

        w0 = _t(L['msg0']['ws']['w'])
        msgs, msgv = pl.pallas_call(
            _edge_layer_body,
            grid=(e // EB,),
            out_shape=[jax.ShapeDtypeStruct((e, WS), F32)] * 2,
            in_specs=[
                pl.BlockSpec((EB, WG // 2), lambda i: (i, 0)),
                pl.BlockSpec((EB, WG // 2), lambda i: (i, 0)),
                pl.BlockSpec((EB, 40), lambda i: (i, 0)),
                _full((33, 33)), _full((32, 100)), _full((33, 100)),
                _full((1, 100)), _full((33, 16)), _full((100, 16)), _full((1, 16)),
                _full((16, 16)), _full((116, 100)), _full((1, 100)),
                _full((16, 16)), _full((100, 16)), _full((1, 16)),
                _full((16, 16)), _full((116, 100)), _full((1, 100)),
                _full((16, 16)), _full((100, 16)), _full((1, 16)),
            ],
            out_specs=[pl.BlockSpec((EB, WS), lambda i: (i, 0))] * 2,
        )(gs, gd, ef,
          _t(L['msg0']['wh']), w0[100:132, :], w0[232:265, :],
          _row(L['msg0']['ws']['b']), _t(L['msg0']['wv']),
          _t(L['msg0']['wsv']['w']), _row(L['msg0']['wsv']['b']),
          _t(L['msg1']['wh']), _t(L['msg1']['ws']['w']), _row(L['msg1']['ws']['b']),
          _t(L['msg1']['wv']), _t(L['msg1']['wsv']['w']), _row(L['msg1']['wsv']['b']),
          _t(L['msg2']['wh']), _t(L['msg2']['ws']['w']), _row(L['msg2']['ws']['b']),
          _t(L['msg2']['wv']), _t(L['msg2']['wsv']['w']), _row(L['msg2']['wsv']['b']))

        parts, partv = _sc_scatter(msgs, msgv, dst3d, zeros)

        fws0T = _t(L['ff0']['ws']['w'])
        fws1T = _t(L['ff1']['ws']['w'])
        common_inputs = (
            parts, partv, sv,
            _row(L['norm0']['g']), _row(L['norm0']['b']),
            _t(L['ff0']['wh']), fws0T[0:100, :], fws0T[100:132, :],
            _row(L['ff0']['ws']['b']), _t(L['ff0']['wv']),
            _t(L['ff0']['wsv']['w']), _row(L['ff0']['wsv']['b']),
            _t(L['ff1']['wh']), fws1T[0:400, :], fws1T[400:432, :],
            _row(L['ff1']['ws']['b']), _t(L['ff1']['wv']),
            _t(L['ff1']['wsv']['w']), _row(L['ff1']['wsv']['b']),
            _row(L['norm1']['g']), _row(L['norm1']['b']),
        )
        common_specs = [
            pl.BlockSpec((2, NB, WS), lambda i: (0, i, 0)),
            pl.BlockSpec((2, NB, WS), lambda i: (0, i, 0)),
            pl.BlockSpec((NB, W), lambda i: (i, 0)),
            _full((1, 100)), _full((1, 100)),
            _full((16, 32)), _full((100, 400)), _full((32, 400)),
            _full((1, 400)), _full((32, 32)), _full((400, 32)), _full((1, 32)),
            _full((32, 32)), _full((400, 100)), _full((32, 100)),
            _full((1, 100)), _full((32, 16)), _full((100, 16)), _full((1, 16)),
            _full((1, 100)), _full((1, 100)),
        ]

        if li < 2:
            Ln = params['layers'][li + 1]
            sv, ts = pl.pallas_call(
                _node_update_body,
                grid=(n // NB,),
                out_shape=[jax.ShapeDtypeStruct((n, W), F32),
                           jax.ShapeDtypeStruct((n, WG // 2), jnp.uint32)],
                in_specs=common_specs + [_full((100, 100)), _full((100, 100))],
                out_specs=[pl.BlockSpec((NB, W), lambda i: (i, 0)),
                           pl.BlockSpec((NB, WG // 2), lambda i: (i, 0))],
            )(*common_inputs,
              _t(Ln['msg0']['ws']['w'][:, 0:100]), _t(Ln['msg0']['ws']['w'][:, 132:232]))
        else:
            owsT = _t(params['Wout_gvp']['ws']['w'])
            out = pl.pallas_call(
                _node_final_body,
                grid=(n // NB,),
                out_shape=jax.ShapeDtypeStruct((n, 100), F32),
                in_specs=common_specs + [
                    _full((1, 100)), _full((1, 100)),
                    _full((16, 16)), _full((100, 100)), _full((16, 100)),
                    _full((1, 100)), _full((100, 100)), _full((1, 100)),
                ],
                out_specs=pl.BlockSpec((NB, 100), lambda i: (i, 0)),
            )(*common_inputs,
              _row(params['Wout_ln']['g']), _row(params['Wout_ln']['b']),
              _t(params['Wout_gvp']['wh']), owsT[0:100, :], owsT[100:116, :],
              _row(params['Wout_gvp']['ws']['b']),
              _t(params['dense']['w']), _row(params['dense']['b']))
    return out


# speedup vs baseline: 1.2482x; 1.2482x over previous
import functools

import jax
import jax.numpy as jnp
from jax import lax
from jax.experimental import pallas as pl
from jax.experimental.pallas import tpu as pltpu
from jax.experimental.pallas import tpu_sc as plsc

F32 = jnp.float32
W = 160
WG = 256
WS = 128
NB = 1000
EB = 4000
EIB = 4000
CH = 80
NCHUNK = 125


def _dot(a, b):
    return jnp.dot(a, b, preferred_element_type=F32)


def _ln(s, g, b):
    mu = jnp.mean(s, axis=1, keepdims=True)
    var = jnp.mean(jnp.square(s - mu), axis=1, keepdims=True)
    return (s - mu) / jnp.sqrt(var + 1e-5) * g + b


def _vnorm3(hx, hy, hz):
    return jnp.sqrt(jnp.clip(hx * hx + hy * hy + hz * hz, 1e-8, None))


def _vln_v(vx, vy, vz):
    vsq = jnp.clip(vx * vx + vy * vy + vz * vz, 1e-8, None)
    den = jnp.sqrt(jnp.mean(vsq, axis=1, keepdims=True))
    return vx / den, vy / den, vz / den


def _pack_rows(row):
    u = lax.bitcast_convert_type(row, jnp.uint32) + jnp.uint32(0x8000)
    return (u[:, 0:128] >> 16) | (u[:, 128:256] & jnp.uint32(0xFFFF0000))


def _unpack_rows(w):
    lo = lax.bitcast_convert_type(w << 16, F32)
    hi = lax.bitcast_convert_type(w & jnp.uint32(0xFFFF0000), F32)
    return lo, hi


def _node_init_body(s6_ref, v9_ref, g_ref, b_ref, whT_ref, wsTs_ref, wsTv_ref,
                    bs_ref, wvT_ref, wsvT_ref, bsv_ref, asrc_ref, adst_ref,
                    sv_ref, ts_ref):
    s = s6_ref[...]
    vx, vy, vz = v9_ref[:, 0:3], v9_ref[:, 3:6], v9_ref[:, 6:9]
    s = _ln(s, g_ref[...], b_ref[...])
    vx, vy, vz = _vln_v(vx, vy, vz)
    whT = whT_ref[...]
    hx, hy, hz = _dot(vx, whT), _dot(vy, whT), _dot(vz, whT)
    vn = _vnorm3(hx, hy, hz)
    spre = _dot(s, wsTs_ref[...]) + _dot(vn, wsTv_ref[...]) + bs_ref[...]
    gate = jax.nn.sigmoid(_dot(spre, wsvT_ref[...]) + bsv_ref[...])
    wvT = wvT_ref[...]
    ox, oy, oz = _dot(hx, wvT) * gate, _dot(hy, wvT) * gate, _dot(hz, wvT) * gate
    zpad = jnp.zeros((spre.shape[0], W - 148), F32)
    sv_ref[...] = jnp.concatenate([spre, ox, oy, oz, zpad], axis=1)
    gpad = jnp.zeros((spre.shape[0], WG - 248), F32)
    ts_ref[...] = _pack_rows(jnp.concatenate(
        [_dot(spre, asrc_ref[...]), _dot(spre, adst_ref[...]), ox, oy, oz, gpad],
        axis=1))


def _edge_init_body(es_ref, ev_ref, g_ref, b_ref, wh_ref, wsTs_ref, wsTv_ref,
                    bs_ref, wv_ref, wsvT_ref, bsv_ref, ef_ref):
    s = _ln(es_ref[...], g_ref[...], b_ref[...])
    vx, vy, vz = ev_ref[:, 0:1], ev_ref[:, 1:2], ev_ref[:, 2:3]
    vx, vy, vz = _vln_v(vx, vy, vz)
    wh00 = wh_ref[0, 0]
    hx, hy, hz = vx * wh00, vy * wh00, vz * wh00
    vn = _vnorm3(hx, hy, hz)
    spre = _dot(s, wsTs_ref[...]) + vn * wsTv_ref[...] + bs_ref[...]
    gate = jax.nn.sigmoid(_dot(spre, wsvT_ref[...]) + bsv_ref[...])
    wv00 = wv_ref[0, 0]
    ox, oy, oz = hx * wv00 * gate, hy * wv00 * gate, hz * wv00 * gate
    zpad = jnp.zeros((spre.shape[0], 5), F32)
    ef_ref[...] = jnp.concatenate([spre, ox, oy, oz, zpad], axis=1)


def _edge_layer_body(gs_ref, gd_ref, ef_ref,
                     wh0T_ref, wes_ref, wvn_ref, b0_ref, wv0T_ref, wsv0T_ref, bsv0_ref,
                     wh1T_ref, ws1T_ref, b1_ref, wv1T_ref, wsv1T_ref, bsv1_ref,
                     wh2T_ref, ws2T_ref, b2_ref, wv2T_ref, wsv2T_ref, bsv2_ref,
                     msgs_ref, msgv_ref):
    glo_s, ghi_s = _unpack_rows(gs_ref[...])
    glo_d, ghi_d = _unpack_rows(gd_ref[...])
    ef = ef_ref[...]
    ps = glo_s[:, 0:100]
    pd = jnp.concatenate([glo_d[:, 100:128], ghi_d[:, 0:72]], axis=1)
    vsx, vsy, vsz = ghi_s[:, 72:88], ghi_s[:, 88:104], ghi_s[:, 104:120]
    vdx, vdy, vdz = ghi_d[:, 72:88], ghi_d[:, 88:104], ghi_d[:, 104:120]
    es = ef[:, 0:32]
    evx, evy, evz = ef[:, 32:33], ef[:, 33:34], ef[:, 34:35]
    wh0T = wh0T_ref[...]
    a, bb, c = wh0T[0:16, :], wh0T[16:17, :], wh0T[17:33, :]
    hx = _dot(vsx, a) + _dot(evx, bb) + _dot(vdx, c)
    hy = _dot(vsy, a) + _dot(evy, bb) + _dot(vdy, c)
    hz = _dot(vsz, a) + _dot(evz, bb) + _dot(vdz, c)
    vn = _vnorm3(hx, hy, hz)
    s0 = ps + pd + _dot(es, wes_ref[...]) + _dot(vn, wvn_ref[...]) + b0_ref[...]
    g0 = jax.nn.sigmoid(_dot(s0, wsv0T_ref[...]) + bsv0_ref[...])
    wv0T = wv0T_ref[...]
    ux, uy, uz = _dot(hx, wv0T) * g0, _dot(hy, wv0T) * g0, _dot(hz, wv0T) * g0
    m = jax.nn.relu(s0)
    wh1T = wh1T_ref[...]
    hx, hy, hz = _dot(ux, wh1T), _dot(uy, wh1T), _dot(uz, wh1T)
    vn = _vnorm3(hx, hy, hz)
    ws1T = ws1T_ref[...]
    s1 = _dot(m, ws1T[0:100, :]) + _dot(vn, ws1T[100:116, :]) + b1_ref[...]
    g1 = jax.nn.sigmoid(_dot(s1, wsv1T_ref[...]) + bsv1_ref[...])
    wv1T = wv1T_ref[...]
    ux, uy, uz = _dot(hx, wv1T) * g1, _dot(hy, wv1T) * g1, _dot(hz, wv1T) * g1
    m = jax.nn.relu(s1)
    wh2T = wh2T_ref[...]
    hx, hy, hz = _dot(ux, wh2T), _dot(uy, wh2T), _dot(uz, wh2T)
    vn = _vnorm3(hx, hy, hz)
    ws2T = ws2T_ref[...]
    s2 = _dot(m, ws2T[0:100, :]) + _dot(vn, ws2T[100:116, :]) + b2_ref[...]
    g2 = jax.nn.sigmoid(_dot(s2, wsv2T_ref[...]) + bsv2_ref[...])
    wv2T = wv2T_ref[...]
    ux, uy, uz = _dot(hx, wv2T) * g2, _dot(hy, wv2T) * g2, _dot(hz, wv2T) * g2
    n = s2.shape[0]
    msgs_ref[...] = jnp.concatenate(
        [s2, jnp.ones((n, 1), F32), jnp.zeros((n, 27), F32)], axis=1)
    msgv_ref[...] = jnp.concatenate(
        [ux, uy, uz, jnp.zeros((n, 80), F32)], axis=1)


def _node_update_math(parts, partv, sv, g0_ref, c0_ref,
                      fwh0T_ref, fws0Ts_ref, fws0Tv_ref, fb0_ref, fwv0T_ref,
                      fwsv0T_ref, fbsv0_ref,
                      fwh1T_ref, fws1Ts_ref, fws1Tv_ref, fb1_ref, fwv1T_ref,
                      fwsv1T_ref, fbsv1_ref, g1_ref, c1_ref):
    aggs = parts[0] + parts[1]
    aggv = partv[0] + partv[1]
    cnt = jnp.clip(aggs[:, 100:101], 1.0, None)
    s = sv[:, 0:100] + aggs[:, 0:100] / cnt
    vx = sv[:, 100:116] + aggv[:, 0:16] / cnt
    vy = sv[:, 116:132] + aggv[:, 16:32] / cnt
    vz = sv[:, 132:148] + aggv[:, 32:48] / cnt
    s = _ln(s, g0_ref[...], c0_ref[...])
    vx, vy, vz = _vln_v(vx, vy, vz)
    fwh0T = fwh0T_ref[...]
    hx, hy, hz = _dot(vx, fwh0T), _dot(vy, fwh0T), _dot(vz, fwh0T)
    vn = _vnorm3(hx, hy, hz)
    f0 = _dot(s, fws0Ts_ref[...]) + _dot(vn, fws0Tv_ref[...]) + fb0_ref[...]
    gate = jax.nn.sigmoid(_dot(f0, fwsv0T_ref[...]) + fbsv0_ref[...])
    fwv0T = fwv0T_ref[...]
    cx, cy, cz = _dot(hx, fwv0T) * gate, _dot(hy, fwv0T) * gate, _dot(hz, fwv0T) * gate
    f0 = jax.nn.relu(f0)
    fwh1T = fwh1T_ref[...]
    hx, hy, hz = _dot(cx, fwh1T), _dot(cy, fwh1T), _dot(cz, fwh1T)
    vn = _vnorm3(hx, hy, hz)
    f1 = _dot(f0, fws1Ts_ref[...]) + _dot(vn, fws1Tv_ref[...]) + fb1_ref[...]
    gate = jax.nn.sigmoid(_dot(f1, fwsv1T_ref[...]) + fbsv1_ref[...])
    fwv1T = fwv1T_ref[...]
    dx, dy, dz = _dot(hx, fwv1T) * gate, _dot(hy, fwv1T) * gate, _dot(hz, fwv1T) * gate
    s = _ln(s + f1, g1_ref[...], c1_ref[...])
    vx, vy, vz = _vln_v(vx + dx, vy + dy, vz + dz)
    return s, vx, vy, vz


def _node_update_body(parts_ref, partv_ref, sv_ref, g0_ref, c0_ref,
                      fwh0T_ref, fws0Ts_ref, fws0Tv_ref, fb0_ref, fwv0T_ref,
                      fwsv0T_ref, fbsv0_ref,
                      fwh1T_ref, fws1Ts_ref, fws1Tv_ref, fb1_ref, fwv1T_ref,
                      fwsv1T_ref, fbsv1_ref, g1_ref, c1_ref,
                      asrc_ref, adst_ref, sv2_ref, ts_ref):
    s, vx, vy, vz = _node_update_math(
        parts_ref[...], partv_ref[...], sv_ref[...], g0_ref, c0_ref,
        fwh0T_ref, fws0Ts_ref, fws0Tv_ref, fb0_ref, fwv0T_ref, fwsv0T_ref,
        fbsv0_ref, fwh1T_ref, fws1Ts_ref, fws1Tv_ref, fb1_ref, fwv1T_ref,
        fwsv1T_ref, fbsv1_ref, g1_ref, c1_ref)
    zpad = jnp.zeros((s.shape[0], W - 148), F32)
    sv2_ref[...] = jnp.concatenate([s, vx, vy, vz, zpad], axis=1)
    gpad = jnp.zeros((s.shape[0], WG - 248), F32)
    ts_ref[...] = _pack_rows(jnp.concatenate(
        [_dot(s, asrc_ref[...]), _dot(s, adst_ref[...]), vx, vy, vz, gpad], axis=1))


def _node_final_body(parts_ref, partv_ref, sv_ref, g0_ref, c0_ref,
                     fwh0T_ref, fws0Ts_ref, fws0Tv_ref, fb0_ref, fwv0T_ref,
                     fwsv0T_ref, fbsv0_ref,
                     fwh1T_ref, fws1Ts_ref, fws1Tv_ref, fb1_ref, fwv1T_ref,
                     fwsv1T_ref, fbsv1_ref, g1_ref, c1_ref,
                     og_ref, oc_ref, owhT_ref, owsTs_ref, owsTv_ref, ob_ref,
                     dwT_ref, db_ref, out_ref):
    s, vx, vy, vz = _node_update_math(
        parts_ref[...], partv_ref[...], sv_ref[...], g0_ref, c0_ref,
        fwh0T_ref, fws0Ts_ref, fws0Tv_ref, fb0_ref, fwv0T_ref, fwsv0T_ref,
        fbsv0_ref, fwh1T_ref, fws1Ts_ref, fws1Tv_ref, fb1_ref, fwv1T_ref,
        fwsv1T_ref, fbsv1_ref, g1_ref, c1_ref)
    s = _ln(s, og_ref[...], oc_ref[...])
    vx, vy, vz = _vln_v(vx, vy, vz)
    owhT = owhT_ref[...]
    hx, hy, hz = _dot(vx, owhT), _dot(vy, owhT), _dot(vz, owhT)
    vn = _vnorm3(hx, hy, hz)
    s = jax.nn.relu(_dot(s, owsTs_ref[...]) + _dot(vn, owsTv_ref[...]) + ob_ref[...])
    out_ref[...] = jax.nn.relu(_dot(s, dwT_ref[...]) + db_ref[...])


def _sc_gather(ts, src3d, dst3d):
    n_edges = src3d.shape[0] * src3d.shape[1] * src3d.shape[2]
    per_w = NCHUNK * CH
    mesh = plsc.VectorSubcoreMesh(core_axis_name="c", subcore_axis_name="s")

    @functools.partial(
        pl.kernel,
        out_type=[jax.ShapeDtypeStruct((n_edges, WG // 2), jnp.uint32),
                  jax.ShapeDtypeStruct((n_edges, WG // 2), jnp.uint32)],
        mesh=mesh,
        scratch_types=[pltpu.VMEM((NCHUNK, CH), jnp.int32),
                       pltpu.VMEM((NCHUNK, CH), jnp.int32),
                       pltpu.VMEM((CH, WG // 2), jnp.uint32),
                       pltpu.VMEM((CH, WG // 2), jnp.uint32),
                       pltpu.SemaphoreType.DMA,
                       pltpu.SemaphoreType.DMA],
    )
    def k(ts_hbm, src_hbm, dst_hbm, gs_hbm, gd_hbm,
          isrc, idst, bufs, bufd, sems, semd):
        wid = lax.axis_index("s") * 2 + lax.axis_index("c")
        base = wid * per_w
        pltpu.sync_copy(src_hbm.at[wid], isrc)
        pltpu.sync_copy(dst_hbm.at[wid], idst)

        @pl.loop(0, NCHUNK)
        def _(j):
            cs = pltpu.async_copy(ts_hbm.at[isrc.at[j]], bufs, sems)
            cd = pltpu.async_copy(ts_hbm.at[idst.at[j]], bufd, semd)
            cs.wait()
            pltpu.sync_copy(bufs, gs_hbm.at[pl.ds(base + j * CH, CH)])
            cd.wait()
            pltpu.sync_copy(bufd, gd_hbm.at[pl.ds(base + j * CH, CH)])

    return k(ts, src3d, dst3d)


def _sc_scatter(msgs, msgv, dst3d, zeros):
    n_nodes = zeros.shape[0]
    per_w = NCHUNK * CH
    rows_per_tile = n_nodes // 10
    mesh = plsc.VectorSubcoreMesh(core_axis_name="c", subcore_axis_name="s")

    @functools.partial(
        pl.kernel,
        out_type=[jax.ShapeDtypeStruct((2, n_nodes, WS), F32),
                  jax.ShapeDtypeStruct((2, n_nodes, WS), F32)],
        mesh=mesh,
        scratch_types=[pltpu.VMEM((NCHUNK, CH), jnp.int32),
                       pltpu.VMEM((CH, WS), F32),
                       pltpu.VMEM_SHARED((n_nodes, WS), F32),
                       pltpu.SemaphoreType.DMA],
    )
    def k(msgs_hbm, msgv_hbm, dst_hbm, zero_hbm, outs_hbm, outv_hbm,
          idxv, mbuf, accum, sem):
        cid = lax.axis_index("c")
        sid = lax.axis_index("s")
        wid = sid * 2 + cid
        rows = pl.ds(sid * rows_per_tile, rows_per_tile)

        @pl.when(sid < 10)
        def _():
            pltpu.sync_copy(zero_hbm.at[rows], accum.at[rows])

        plsc.subcore_barrier()
        pltpu.sync_copy(dst_hbm.at[wid], idxv)

        @pl.loop(0, NCHUNK)
        def _(j):
            pltpu.sync_copy(msgs_hbm.at[pl.ds(wid * per_w + j * CH, CH)], mbuf)
            pltpu.sync_copy(mbuf, accum.at[idxv.at[j]], add=True)

        plsc.subcore_barrier()

        @pl.when(sid < 10)
        def _():
            pltpu.sync_copy(accum.at[rows], outs_hbm.at[cid].at[rows])
            pltpu.sync_copy(zero_hbm.at[rows], accum.at[rows])

        plsc.subcore_barrier()

        @pl.loop(0, NCHUNK)
        def _(j):
            pltpu.sync_copy(msgv_hbm.at[pl.ds(wid * per_w + j * CH, CH)], mbuf)
            pltpu.sync_copy(mbuf, accum.at[idxv.at[j]], add=True)

        plsc.subcore_barrier()

        @pl.when(sid < 10)
        def _():
            pltpu.sync_copy(accum.at[rows], outv_hbm.at[cid].at[rows])

    return k(msgs, msgv, dst3d, zeros)


def _t(p):
    return jnp.asarray(p, F32).T


def _row(p):
    return jnp.asarray(p, F32).reshape(1, -1)


def _full(shape):
    return pl.BlockSpec(shape, lambda i: tuple(0 for _ in shape))


def kernel(h_V_s, h_V_v, edge_index, h_E_s, h_E_v, params):
    n = h_V_s.shape[0]
    e = h_E_s.shape[0]
    src3d = edge_index[0].astype(jnp.int32).reshape(32, NCHUNK, CH)
    dst3d = edge_index[1].astype(jnp.int32).reshape(32, NCHUNK, CH)
    v9 = jnp.swapaxes(h_V_v, 1, 2).reshape(n, 9)
    ev3 = h_E_v.reshape(e, 3)
    zeros = jnp.zeros((n, WS), F32)
    L0 = params['layers'][0]

    gvp = params['Wv_gvp']
    wsT = _t(gvp['ws']['w'])
    sv, ts = pl.pallas_call(
        _node_init_body,
        grid=(n // NB,),
        out_shape=[jax.ShapeDtypeStruct((n, W), F32),
                   jax.ShapeDtypeStruct((n, WG // 2), jnp.uint32)],
        in_specs=[
            pl.BlockSpec((NB, 6), lambda i: (i, 0)),
            pl.BlockSpec((NB, 9), lambda i: (i, 0)),
            _full((1, 6)), _full((1, 6)),
            _full((3, 16)),
            _full((6, 100)), _full((16, 100)), _full((1, 100)),
            _full((16, 16)), _full((100, 16)), _full((1, 16)),
            _full((100, 100)), _full((100, 100)),
        ],
        out_specs=[pl.BlockSpec((NB, W), lambda i: (i, 0)),
                   pl.BlockSpec((NB, WG // 2), lambda i: (i, 0))],
    )(h_V_s.astype(F32), v9.astype(F32),
      _row(params['Wv_ln']['g']), _row(params['Wv_ln']['b']),
      _t(gvp['wh']), wsT[0:6, :], wsT[6:22, :], _row(gvp['ws']['b']),
      _t(gvp['wv']), _t(gvp['wsv']['w']), _row(gvp['wsv']['b']),
      _t(L0['msg0']['ws']['w'][:, 0:100]), _t(L0['msg0']['ws']['w'][:, 132:232]))

    gvp = params['We_gvp']
    wsT = _t(gvp['ws']['w'])
    ef = pl.pallas_call(
        _edge_init_body,
        grid=(e // EIB,),
        out_shape=jax.ShapeDtypeStruct((e, 40), F32),
        in_specs=[
            pl.BlockSpec((EIB, 32), lambda i: (i, 0)),
            pl.BlockSpec((EIB, 3), lambda i: (i, 0)),
            _full((1, 32)), _full((1, 32)),
            _full((1, 1)),
            _full((32, 32)), _full((1, 32)), _full((1, 32)),
            _full((1, 1)), _full((32, 1)), _full((1, 1)),
        ],
        out_specs=pl.BlockSpec((EIB, 40), lambda i: (i, 0)),
    )(h_E_s.astype(F32), ev3.astype(F32),
      _row(params['We_ln']['g']), _row(params['We_ln']['b']),
      jnp.asarray(gvp['wh'], F32), wsT[0:32, :], wsT[32:33, :],
      _row(gvp['ws']['b']), jnp.asarray(gvp['wv'], F32),
      _t(gvp['wsv']['w']), _row(gvp['wsv']['b']))

    for li, L in enumerate(params['layers']):
        gs, gd = _sc_gather(ts, src3d, dst3d)

        w0 = _t(L['msg0']['ws']['w'])
        msgs, msgv = pl.pallas_call(
            _edge_layer_body,
            grid=(e // EB,),
            out_shape=[jax.ShapeDtypeStruct((e, WS), F32)] * 2,
            in_specs=[
                pl.BlockSpec((EB, WG // 2), lambda i: (i, 0)),
                pl.BlockSpec((EB, WG // 2), lambda i: (i, 0)),
                pl.BlockSpec((EB, 40), lambda i: (i, 0)),
                _full((33, 33)), _full((32, 100)), _full((33, 100)),
                _full((1, 100)), _full((33, 16)), _full((100, 16)), _full((1, 16)),
                _full((16, 16)), _full((116, 100)), _full((1, 100)),
                _full((16, 16)), _full((100, 16)), _full((1, 16)),
                _full((16, 16)), _full((116, 100)), _full((1, 100)),
                _full((16, 16)), _full((100, 16)), _full((1, 16)),
            ],
            out_specs=[pl.BlockSpec((EB, WS), lambda i: (i, 0))] * 2,
        )(gs, gd, ef,
          _t(L['msg0']['wh']), w0[100:132, :], w0[232:265, :],
          _row(L['msg0']['ws']['b']), _t(L['msg0']['wv']),
          _t(L['msg0']['wsv']['w']), _row(L['msg0']['wsv']['b']),
          _t(L['msg1']['wh']), _t(L['msg1']['ws']['w']), _row(L['msg1']['ws']['b']),
          _t(L['msg1']['wv']), _t(L['msg1']['wsv']['w']), _row(L['msg1']['wsv']['b']),
          _t(L['msg2']['wh']), _t(L['msg2']['ws']['w']), _row(L['msg2']['ws']['b']),
          _t(L['msg2']['wv']), _t(L['msg2']['wsv']['w']), _row(L['msg2']['wsv']['b']))

        parts, partv = _sc_scatter(msgs, msgv, dst3d, zeros)

        fws0T = _t(L['ff0']['ws']['w'])
        fws1T = _t(L['ff1']['ws']['w'])
        common_inputs = (
            parts, partv, sv,
            _row(L['norm0']['g']), _row(L['norm0']['b']),
            _t(L['ff0']['wh']), fws0T[0:100, :], fws0T[100:132, :],
            _row(L['ff0']['ws']['b']), _t(L['ff0']['wv']),
            _t(L['ff0']['wsv']['w']), _row(L['ff0']['wsv']['b']),
            _t(L['ff1']['wh']), fws1T[0:400, :], fws1T[400:432, :],
            _row(L['ff1']['ws']['b']), _t(L['ff1']['wv']),
            _t(L['ff1']['wsv']['w']), _row(L['ff1']['wsv']['b']),
            _row(L['norm1']['g']), _row(L['norm1']['b']),
        )
        common_specs = [
            pl.BlockSpec((2, NB, WS), lambda i: (0, i, 0)),
            pl.BlockSpec((2, NB, WS), lambda i: (0, i, 0)),
            pl.BlockSpec((NB, W), lambda i: (i, 0)),
            _full((1, 100)), _full((1, 100)),
            _full((16, 32)), _full((100, 400)), _full((32, 400)),
            _full((1, 400)), _full((32, 32)), _full((400, 32)), _full((1, 32)),
            _full((32, 32)), _full((400, 100)), _full((32, 100)),
            _full((1, 100)), _full((32, 16)), _full((100, 16)), _full((1, 16)),
            _full((1, 100)), _full((1, 100)),
        ]

        if li < 2:
            Ln = params['layers'][li + 1]
            sv, ts = pl.pallas_call(
                _node_update_body,
                grid=(n // NB,),
                out_shape=[jax.ShapeDtypeStruct((n, W), F32),
                           jax.ShapeDtypeStruct((n, WG // 2), jnp.uint32)],
                in_specs=common_specs + [_full((100, 100)), _full((100, 100))],
                out_specs=[pl.BlockSpec((NB, W), lambda i: (i, 0)),
                           pl.BlockSpec((NB, WG // 2), lambda i: (i, 0))],
            )(*common_inputs,
              _t(Ln['msg0']['ws']['w'][:, 0:100]), _t(Ln['msg0']['ws']['w'][:, 132:232]))
        else:
            owsT = _t(params['Wout_gvp']['ws']['w'])
            out = pl.pallas_call(
                _node_final_body,
                grid=(n // NB,),
                out_shape=jax.ShapeDtypeStruct((n, 100), F32),
                in_specs=common_specs + [
                    _full((1, 100)), _full((1, 100)),
                    _full((16, 16)), _full((100, 100)), _full((16, 100)),
                    _full((1, 100)), _full((100, 100)), _full((1, 100)),
                ],
                out_specs=pl.BlockSpec((NB, 100), lambda i: (i, 0)),
            )(*common_inputs,
              _row(params['Wout_ln']['g']), _row(params['Wout_ln']['b']),
              _t(params['Wout_gvp']['wh']), owsT[0:100, :], owsT[100:116, :],
              _row(params['Wout_gvp']['ws']['b']),
              _t(params['dense']['w']), _row(params['dense']['b']))
    return out
